# baseline (speedup 1.0000x reference)
import functools

import jax
import jax.numpy as jnp
from jax import lax
from jax.experimental import pallas as pl
from jax.experimental.pallas import tpu as pltpu, tpu_sc as plsc

N_NODES = 10000
N_EDGES = 320000
D = 128
DEPTH = 3

NC = 2
NS = 16
NW = NC * NS
CHUNK = 80
CPT = 128
NCHUNKS_PAD = CPT * NW
NSTAGES = 8
STG = CPT // NSTAGES
N_DUMMY = 64
M_ROWS = N_NODES + N_DUMMY
STRIDE = 624
STRIPE = 640


def _mp_body(h_hbm, src_hbm, dst_hbm, z_hbm, out_hbm,
             src_v, dst_v, rows0, rows1, rows2, rows3,
             m_sh, g0, g1, g2, g3, s0, s1, s2, s3):
    c = lax.axis_index("c")
    s = lax.axis_index("s")
    w = c * NS + s

    pltpu.sync_copy(z_hbm.at[pl.ds(s * STRIDE, STRIPE)],
                    m_sh.at[pl.ds(s * STRIDE, STRIPE)])

    @pl.when(s == 0)
    def _():
        pltpu.sync_copy(z_hbm.at[pl.ds(N_NODES, N_DUMMY)],
                        m_sh.at[pl.ds(N_NODES, N_DUMMY)])

    plsc.subcore_barrier()

    base = CPT * w
    bufs = (rows0, rows1, rows2, rows3)
    gsems = (g0, g1, g2, g3)
    ssems = (s0, s1, s2, s3)

    def start_gather(j, buf, gsem):
        pltpu.async_copy(h_hbm.at[src_v.at[j]], buf, gsem)

    def wait_gather(buf, gsem):
        pltpu.make_async_copy(h_hbm.at[pl.ds(0, CHUNK)], buf, gsem).wait()

    def start_scatter(j, buf, ssem):
        pltpu.async_copy(buf, m_sh.at[dst_v.at[j]], ssem, add=True)

    def wait_scatter(buf, ssem):
        pltpu.make_async_copy(buf, m_sh.at[pl.ds(0, CHUNK)], ssem).wait()

    def stage_body(k, carry):
        pltpu.sync_copy(src_hbm.at[pl.ds(base + STG * k, STG)], src_v)
        pltpu.sync_copy(dst_hbm.at[pl.ds(base + STG * k, STG)], dst_v)
        start_gather(0, rows0, g0)
        start_gather(1, rows1, g1)

        def quad(q, c2):
            j = 4 * q

            @pl.when(q > 0)
            def _():
                wait_scatter(rows2, s2)

            start_gather(j + 2, rows2, g2)
            wait_gather(rows0, g0)
            start_scatter(j, rows0, s0)

            @pl.when(q > 0)
            def _():
                wait_scatter(rows3, s3)

            start_gather(j + 3, rows3, g3)
            wait_gather(rows1, g1)
            start_scatter(j + 1, rows1, s1)

            @pl.when(q < STG // 4 - 1)
            def _():
                wait_scatter(rows0, s0)
                start_gather(j + 4, rows0, g0)

            wait_gather(rows2, g2)
            start_scatter(j + 2, rows2, s2)

            @pl.when(q < STG // 4 - 1)
            def _():
                wait_scatter(rows1, s1)
                start_gather(j + 5, rows1, g1)

            wait_gather(rows3, g3)
            start_scatter(j + 3, rows3, s3)
            return c2

        lax.fori_loop(0, STG // 4, quad, 0)
        wait_scatter(rows0, s0)
        wait_scatter(rows1, s1)
        wait_scatter(rows2, s2)
        wait_scatter(rows3, s3)
        return carry

    lax.fori_loop(0, NSTAGES, stage_body, 0)

    plsc.subcore_barrier()
    pltpu.sync_copy(m_sh.at[pl.ds(s * STRIDE, STRIPE)],
                    out_hbm.at[c, pl.ds(s * STRIDE, STRIPE)])


_message_pass = functools.partial(
    pl.kernel,
    out_type=jax.ShapeDtypeStruct((NC, N_NODES, D), jnp.float32),
    mesh=plsc.VectorSubcoreMesh(core_axis_name="c", subcore_axis_name="s"),
    scratch_types=(
        [pltpu.VMEM((STG, CHUNK), jnp.int32)] * 2
        + [pltpu.VMEM((CHUNK, D), jnp.float32)] * 4
        + [pltpu.VMEM_SHARED((M_ROWS, D), jnp.float32)]
        + [pltpu.SemaphoreType.DMA] * 8
    ),
)(_mp_body)


ROWS_BLK = 1000
GRID = N_NODES // ROWS_BLK


def _pre_body(h_ref, A_ref, b_ref, out_ref):
    out_ref[...] = (jnp.dot(h_ref[...], A_ref[...],
                            preferred_element_type=jnp.float32) + b_ref[...])


def _update_body(p_ref, m0_ref, m1_ref, B_ref, out_ref):
    m = m0_ref[...] + m1_ref[...]
    acc = p_ref[...] + jnp.dot(m, B_ref[...],
                               preferred_element_type=jnp.float32)
    out_ref[...] = jnp.maximum(acc, 0.0)


def _final_body(p_ref, m0_ref, m1_ref, B_ref, nnw_ref, nnb_ref,
                out_ref, acc_ref):
    i = pl.program_id(0)
    m = m0_ref[...] + m1_ref[...]
    acc = p_ref[...] + jnp.dot(m, B_ref[...],
                               preferred_element_type=jnp.float32)
    h_new = jnp.maximum(acc, 0.0)
    part = jnp.sum(h_new, axis=0, keepdims=True)

    @pl.when(i == 0)
    def _():
        acc_ref[...] = part

    @pl.when(i > 0)
    def _():
        acc_ref[...] = acc_ref[...] + part

    @pl.when(i == GRID - 1)
    def _():
        out_ref[...] = (jnp.sum(acc_ref[...] * nnw_ref[...])
                        + nnb_ref[0, 0]).reshape(1, 1)


def _row_spec():
    return pl.BlockSpec((ROWS_BLK, D), lambda i: (i, 0))


def _full_spec(shape):
    return pl.BlockSpec(shape, lambda i: (0,) * len(shape))


_pre = pl.pallas_call(
    _pre_body,
    grid=(GRID,),
    in_specs=[_row_spec(), _full_spec((D, D)), _full_spec((1, D))],
    out_specs=_row_spec(),
    out_shape=jax.ShapeDtypeStruct((N_NODES, D), jnp.float32),
)

_update = pl.pallas_call(
    _update_body,
    grid=(GRID,),
    in_specs=[_row_spec(), _row_spec(), _row_spec(),
              _full_spec((D, D))],
    out_specs=_row_spec(),
    out_shape=jax.ShapeDtypeStruct((N_NODES, D), jnp.float32),
)

_update_final = pl.pallas_call(
    _final_body,
    grid=(GRID,),
    in_specs=[_row_spec(), _row_spec(), _row_spec(),
              _full_spec((D, D)),
              _full_spec((1, D)), _full_spec((1, 1))],
    out_specs=_full_spec((1, 1)),
    out_shape=jax.ShapeDtypeStruct((1, 1), jnp.float32),
    scratch_shapes=[pltpu.VMEM((1, D), jnp.float32)],
)


@jax.jit
def kernel(x, edge_index, U_w, U_b, NN_w, NN_b):
    n_pad = NCHUNKS_PAD * CHUNK - N_EDGES
    pad_src = (jnp.arange(n_pad, dtype=jnp.int32) % N_NODES).reshape(-1, CHUNK)
    pad_dst = (N_NODES
               + jnp.arange(n_pad, dtype=jnp.int32) % N_DUMMY).reshape(-1, CHUNK)
    src = jnp.concatenate(
        [edge_index[0].astype(jnp.int32).reshape(-1, CHUNK), pad_src])
    dst = jnp.concatenate(
        [edge_index[1].astype(jnp.int32).reshape(-1, CHUNK), pad_dst])
    A = U_w[:, :D].T
    B = U_w[:, D:].T
    b = U_b.reshape(1, D)
    nnw = NN_w.reshape(1, D)
    nnb = NN_b.reshape(1, 1)
    zeros = jnp.zeros((M_ROWS, D), jnp.float32)

    h = x
    for d in range(DEPTH):
        mp = _message_pass(h, src, dst, zeros)
        P = _pre(h, A, b)
        if d < DEPTH - 1:
            h = _update(P, mp[0], mp[1], B)
        else:
            out = _update_final(P, mp[0], mp[1], B, nnw, nnb)
    return out.reshape(1)


# speedup vs baseline: 1.0511x; 1.0511x over previous
import functools

import jax
import jax.numpy as jnp
from jax import lax
from jax.experimental import pallas as pl
from jax.experimental.pallas import tpu as pltpu, tpu_sc as plsc

N_NODES = 10000
N_EDGES = 320000
D = 128
DEPTH = 3

NC = 2
NS = 16
NW = NC * NS
CHUNK = 64
CPT = 160
NCHUNKS_PAD = CPT * NW
NSTAGES = 4
STG = CPT // NSTAGES
N_DUMMY = 64
M_ROWS = N_NODES + N_DUMMY
STRIDE = 624
STRIPE = 640


def _mp_body(h_hbm, src_hbm, dst_hbm, z_hbm, out_hbm,
             src_v, dst_v, rows0, rows1, rows2, rows3,
             m_sh, g0, g1, g2, g3, s0, s1, s2, s3):
    c = lax.axis_index("c")
    s = lax.axis_index("s")
    w = c * NS + s

    pltpu.sync_copy(z_hbm.at[pl.ds(s * STRIDE, STRIPE)],
                    m_sh.at[pl.ds(s * STRIDE, STRIPE)])

    @pl.when(s == 0)
    def _():
        pltpu.sync_copy(z_hbm.at[pl.ds(N_NODES, N_DUMMY)],
                        m_sh.at[pl.ds(N_NODES, N_DUMMY)])

    plsc.subcore_barrier()

    base = CPT * w

    def start_gather(j, buf, gsem):
        pltpu.async_copy(h_hbm.at[src_v.at[j]], buf, gsem)

    def wait_gather(buf, gsem):
        pltpu.make_async_copy(h_hbm.at[pl.ds(0, CHUNK)], buf, gsem).wait()

    def start_scatter(j, buf, ssem):
        pltpu.async_copy(buf, m_sh.at[dst_v.at[j]], ssem, add=True)

    def wait_scatter(buf, ssem):
        pltpu.make_async_copy(buf, m_sh.at[pl.ds(0, CHUNK)], ssem).wait()

    def stage_body(k, carry):
        pltpu.sync_copy(src_hbm.at[pl.ds(base + STG * k, STG)], src_v)
        pltpu.sync_copy(dst_hbm.at[pl.ds(base + STG * k, STG)], dst_v)
        start_gather(0, rows0, g0)
        start_gather(1, rows1, g1)

        def quad(q, c2):
            j = 4 * q

            @pl.when(q > 0)
            def _():
                wait_scatter(rows2, s2)

            start_gather(j + 2, rows2, g2)
            wait_gather(rows0, g0)
            start_scatter(j, rows0, s0)

            @pl.when(q > 0)
            def _():
                wait_scatter(rows3, s3)

            start_gather(j + 3, rows3, g3)
            wait_gather(rows1, g1)
            start_scatter(j + 1, rows1, s1)

            @pl.when(q < STG // 4 - 1)
            def _():
                wait_scatter(rows0, s0)
                start_gather(j + 4, rows0, g0)

            wait_gather(rows2, g2)
            start_scatter(j + 2, rows2, s2)

            @pl.when(q < STG // 4 - 1)
            def _():
                wait_scatter(rows1, s1)
                start_gather(j + 5, rows1, g1)

            wait_gather(rows3, g3)
            start_scatter(j + 3, rows3, s3)
            return c2

        lax.fori_loop(0, STG // 4, quad, 0)
        wait_scatter(rows0, s0)
        wait_scatter(rows1, s1)
        wait_scatter(rows2, s2)
        wait_scatter(rows3, s3)
        return carry

    lax.fori_loop(0, NSTAGES, stage_body, 0)

    plsc.subcore_barrier()
    pltpu.sync_copy(m_sh.at[pl.ds(s * STRIDE, STRIPE)],
                    out_hbm.at[c, pl.ds(s * STRIDE, STRIPE)])


_message_pass = functools.partial(
    pl.kernel,
    out_type=jax.ShapeDtypeStruct((NC, N_NODES, D), jnp.float32),
    mesh=plsc.VectorSubcoreMesh(core_axis_name="c", subcore_axis_name="s"),
    scratch_types=(
        [pltpu.VMEM((STG, CHUNK), jnp.int32)] * 2
        + [pltpu.VMEM((CHUNK, D), jnp.float32)] * 4
        + [pltpu.VMEM_SHARED((M_ROWS, D), jnp.float32)]
        + [pltpu.SemaphoreType.DMA] * 8
    ),
)(_mp_body)


ROWS_BLK = 1000
GRID = N_NODES // ROWS_BLK


def _pre_body(h_ref, A_ref, b_ref, out_ref):
    out_ref[...] = (jnp.dot(h_ref[...], A_ref[...],
                            preferred_element_type=jnp.float32) + b_ref[...])


def _update_body(p_ref, m0_ref, m1_ref, B_ref, out_ref):
    m = m0_ref[...] + m1_ref[...]
    acc = p_ref[...] + jnp.dot(m, B_ref[...],
                               preferred_element_type=jnp.float32)
    out_ref[...] = jnp.maximum(acc, 0.0)


def _final_body(p_ref, m0_ref, m1_ref, B_ref, nnw_ref, nnb_ref,
                out_ref, acc_ref):
    i = pl.program_id(0)
    m = m0_ref[...] + m1_ref[...]
    acc = p_ref[...] + jnp.dot(m, B_ref[...],
                               preferred_element_type=jnp.float32)
    h_new = jnp.maximum(acc, 0.0)
    part = jnp.sum(h_new, axis=0, keepdims=True)

    @pl.when(i == 0)
    def _():
        acc_ref[...] = part

    @pl.when(i > 0)
    def _():
        acc_ref[...] = acc_ref[...] + part

    @pl.when(i == GRID - 1)
    def _():
        out_ref[...] = (jnp.sum(acc_ref[...] * nnw_ref[...])
                        + nnb_ref[0, 0]).reshape(1, 1)


def _row_spec():
    return pl.BlockSpec((ROWS_BLK, D), lambda i: (i, 0))


def _full_spec(shape):
    return pl.BlockSpec(shape, lambda i: (0,) * len(shape))


_pre = pl.pallas_call(
    _pre_body,
    grid=(GRID,),
    in_specs=[_row_spec(), _full_spec((D, D)), _full_spec((1, D))],
    out_specs=_row_spec(),
    out_shape=jax.ShapeDtypeStruct((N_NODES, D), jnp.float32),
)

_update = pl.pallas_call(
    _update_body,
    grid=(GRID,),
    in_specs=[_row_spec(), _row_spec(), _row_spec(),
              _full_spec((D, D))],
    out_specs=_row_spec(),
    out_shape=jax.ShapeDtypeStruct((N_NODES, D), jnp.float32),
)

_update_final = pl.pallas_call(
    _final_body,
    grid=(GRID,),
    in_specs=[_row_spec(), _row_spec(), _row_spec(),
              _full_spec((D, D)),
              _full_spec((1, D)), _full_spec((1, 1))],
    out_specs=_full_spec((1, 1)),
    out_shape=jax.ShapeDtypeStruct((1, 1), jnp.float32),
    scratch_shapes=[pltpu.VMEM((1, D), jnp.float32)],
)


@jax.jit
def kernel(x, edge_index, U_w, U_b, NN_w, NN_b):
    n_pad = NCHUNKS_PAD * CHUNK - N_EDGES
    pad_src = (jnp.arange(n_pad, dtype=jnp.int32) % N_NODES).reshape(-1, CHUNK)
    pad_dst = (N_NODES
               + jnp.arange(n_pad, dtype=jnp.int32) % N_DUMMY).reshape(-1, CHUNK)
    src = jnp.concatenate(
        [edge_index[0].astype(jnp.int32).reshape(-1, CHUNK), pad_src])
    dst = jnp.concatenate(
        [edge_index[1].astype(jnp.int32).reshape(-1, CHUNK), pad_dst])
    A = U_w[:, :D].T
    B = U_w[:, D:].T
    b = U_b.reshape(1, D)
    nnw = NN_w.reshape(1, D)
    nnb = NN_b.reshape(1, 1)
    zeros = jnp.zeros((M_ROWS, D), jnp.float32)

    h = x
    for d in range(DEPTH):
        mp = _message_pass(h, src, dst, zeros)
        P = _pre(h, A, b)
        if d < DEPTH - 1:
            h = _update(P, mp[0], mp[1], B)
        else:
            out = _update_final(P, mp[0], mp[1], B, nnw, nnb)
    return out.reshape(1)


# speedup vs baseline: 1.0512x; 1.0001x over previous
import functools

import jax
import jax.numpy as jnp
from jax import lax
from jax.experimental import pallas as pl
from jax.experimental.pallas import tpu as pltpu, tpu_sc as plsc

N_NODES = 10000
N_EDGES = 320000
D = 128
DEPTH = 3

NC = 2
NS = 16
NW = NC * NS
CHUNK = 64
CPT = 160
NCHUNKS_PAD = CPT * NW
NSTAGES = 4
STG = CPT // NSTAGES
N_DUMMY = 64
M_ROWS = N_NODES + N_DUMMY
STRIDE = 624
STRIPE = 640


def _mp_body(h_hbm, src_hbm, dst_hbm, z_hbm, out_hbm,
             src_v, dst_v, rows0, rows1, rows2, rows3,
             m_sh, g0, g1, g2, g3, s0, s1, s2, s3):
    c = lax.axis_index("c")
    s = lax.axis_index("s")
    w = c * NS + s

    pltpu.sync_copy(z_hbm.at[pl.ds(s * STRIDE, STRIPE)],
                    m_sh.at[pl.ds(s * STRIDE, STRIPE)])

    @pl.when(s == 0)
    def _():
        pltpu.sync_copy(z_hbm.at[pl.ds(N_NODES, N_DUMMY)],
                        m_sh.at[pl.ds(N_NODES, N_DUMMY)])

    base = CPT * w

    pltpu.sync_copy(src_hbm.at[pl.ds(base, STG)], src_v)
    pltpu.sync_copy(dst_hbm.at[pl.ds(base, STG)], dst_v)

    plsc.subcore_barrier()

    def start_gather(j, buf, gsem):
        pltpu.async_copy(h_hbm.at[src_v.at[j]], buf, gsem)

    def wait_gather(buf, gsem):
        pltpu.make_async_copy(h_hbm.at[pl.ds(0, CHUNK)], buf, gsem).wait()

    def start_scatter(j, buf, ssem):
        pltpu.async_copy(buf, m_sh.at[dst_v.at[j]], ssem, add=True)

    def wait_scatter(buf, ssem):
        pltpu.make_async_copy(buf, m_sh.at[pl.ds(0, CHUNK)], ssem).wait()

    def stage_body(k, carry):
        @pl.when(k > 0)
        def _():
            pltpu.sync_copy(src_hbm.at[pl.ds(base + STG * k, STG)], src_v)
            pltpu.sync_copy(dst_hbm.at[pl.ds(base + STG * k, STG)], dst_v)

        start_gather(0, rows0, g0)
        start_gather(1, rows1, g1)

        def quad(q, c2):
            j = 4 * q

            @pl.when(q > 0)
            def _():
                wait_scatter(rows2, s2)

            start_gather(j + 2, rows2, g2)
            wait_gather(rows0, g0)
            start_scatter(j, rows0, s0)

            @pl.when(q > 0)
            def _():
                wait_scatter(rows3, s3)

            start_gather(j + 3, rows3, g3)
            wait_gather(rows1, g1)
            start_scatter(j + 1, rows1, s1)

            @pl.when(q < STG // 4 - 1)
            def _():
                wait_scatter(rows0, s0)
                start_gather(j + 4, rows0, g0)

            wait_gather(rows2, g2)
            start_scatter(j + 2, rows2, s2)

            @pl.when(q < STG // 4 - 1)
            def _():
                wait_scatter(rows1, s1)
                start_gather(j + 5, rows1, g1)

            wait_gather(rows3, g3)
            start_scatter(j + 3, rows3, s3)
            return c2

        lax.fori_loop(0, STG // 4, quad, 0)
        wait_scatter(rows0, s0)
        wait_scatter(rows1, s1)
        wait_scatter(rows2, s2)
        wait_scatter(rows3, s3)
        return carry

    lax.fori_loop(0, NSTAGES, stage_body, 0)

    plsc.subcore_barrier()
    pltpu.sync_copy(m_sh.at[pl.ds(s * STRIDE, STRIPE)],
                    out_hbm.at[c, pl.ds(s * STRIDE, STRIPE)])


_message_pass = functools.partial(
    pl.kernel,
    out_type=jax.ShapeDtypeStruct((NC, N_NODES, D), jnp.float32),
    mesh=plsc.VectorSubcoreMesh(core_axis_name="c", subcore_axis_name="s"),
    scratch_types=(
        [pltpu.VMEM((STG, CHUNK), jnp.int32)] * 2
        + [pltpu.VMEM((CHUNK, D), jnp.float32)] * 4
        + [pltpu.VMEM_SHARED((M_ROWS, D), jnp.float32)]
        + [pltpu.SemaphoreType.DMA] * 8
    ),
)(_mp_body)


ROWS_BLK = 1000
GRID = N_NODES // ROWS_BLK


def _pre_body(h_ref, A_ref, b_ref, out_ref):
    out_ref[...] = (jnp.dot(h_ref[...], A_ref[...],
                            preferred_element_type=jnp.float32) + b_ref[...])


def _update_body(p_ref, m0_ref, m1_ref, B_ref, out_ref):
    m = m0_ref[...] + m1_ref[...]
    acc = p_ref[...] + jnp.dot(m, B_ref[...],
                               preferred_element_type=jnp.float32)
    out_ref[...] = jnp.maximum(acc, 0.0)


def _final_body(p_ref, m0_ref, m1_ref, B_ref, nnw_ref, nnb_ref,
                out_ref, acc_ref):
    i = pl.program_id(0)
    m = m0_ref[...] + m1_ref[...]
    acc = p_ref[...] + jnp.dot(m, B_ref[...],
                               preferred_element_type=jnp.float32)
    h_new = jnp.maximum(acc, 0.0)
    part = jnp.sum(h_new, axis=0, keepdims=True)

    @pl.when(i == 0)
    def _():
        acc_ref[...] = part

    @pl.when(i > 0)
    def _():
        acc_ref[...] = acc_ref[...] + part

    @pl.when(i == GRID - 1)
    def _():
        out_ref[...] = (jnp.sum(acc_ref[...] * nnw_ref[...])
                        + nnb_ref[0, 0]).reshape(1, 1)


def _row_spec():
    return pl.BlockSpec((ROWS_BLK, D), lambda i: (i, 0))


def _full_spec(shape):
    return pl.BlockSpec(shape, lambda i: (0,) * len(shape))


_pre = pl.pallas_call(
    _pre_body,
    grid=(GRID,),
    in_specs=[_row_spec(), _full_spec((D, D)), _full_spec((1, D))],
    out_specs=_row_spec(),
    out_shape=jax.ShapeDtypeStruct((N_NODES, D), jnp.float32),
)

_update = pl.pallas_call(
    _update_body,
    grid=(GRID,),
    in_specs=[_row_spec(), _row_spec(), _row_spec(),
              _full_spec((D, D))],
    out_specs=_row_spec(),
    out_shape=jax.ShapeDtypeStruct((N_NODES, D), jnp.float32),
)

_update_final = pl.pallas_call(
    _final_body,
    grid=(GRID,),
    in_specs=[_row_spec(), _row_spec(), _row_spec(),
              _full_spec((D, D)),
              _full_spec((1, D)), _full_spec((1, 1))],
    out_specs=_full_spec((1, 1)),
    out_shape=jax.ShapeDtypeStruct((1, 1), jnp.float32),
    scratch_shapes=[pltpu.VMEM((1, D), jnp.float32)],
)


@jax.jit
def kernel(x, edge_index, U_w, U_b, NN_w, NN_b):
    n_pad = NCHUNKS_PAD * CHUNK - N_EDGES
    pad_src = (jnp.arange(n_pad, dtype=jnp.int32) % N_NODES).reshape(-1, CHUNK)
    pad_dst = (N_NODES
               + jnp.arange(n_pad, dtype=jnp.int32) % N_DUMMY).reshape(-1, CHUNK)
    src = jnp.concatenate(
        [edge_index[0].astype(jnp.int32).reshape(-1, CHUNK), pad_src])
    dst = jnp.concatenate(
        [edge_index[1].astype(jnp.int32).reshape(-1, CHUNK), pad_dst])
    A = U_w[:, :D].T
    B = U_w[:, D:].T
    b = U_b.reshape(1, D)
    nnw = NN_w.reshape(1, D)
    nnb = NN_b.reshape(1, 1)
    zeros = jnp.zeros((M_ROWS, D), jnp.float32)

    h = x
    for d in range(DEPTH):
        mp = _message_pass(h, src, dst, zeros)
        P = _pre(h, A, b)
        if d < DEPTH - 1:
            h = _update(P, mp[0], mp[1], B)
        else:
            out = _update_final(P, mp[0], mp[1], B, nnw, nnb)
    return out.reshape(1)
